# baseline (speedup 1.0000x reference)
import functools

import jax
import jax.numpy as jnp
import numpy as np
from jax import lax
from jax.experimental import pallas as pl
from jax.experimental.pallas import tpu as pltpu
from jax.experimental.pallas import tpu_sc as plsc

DIM = 16
TRIL = DIM * (DIM + 1) // 2
ROWP = TRIL + 1
OUT_ROW = DIM * DIM
B, S = 1024, 200
N = B * S

NC, NS = 2, 16
NW = NC * NS

VOCAB = 1000000
VPAD = 1000064
NBLK = VPAD // 128
ABLK_PER_W = -(-NBLK // NW)

CHUNK = 128
NCHUNK_TOTAL = N // CHUNK
CHUNK_PER_W = NCHUNK_TOTAL // NW

_ti, _tj = np.tril_indices(DIM)
_pos = (DIM * _ti + _tj).astype(np.int32)
_is_diag = (_ti == _tj)


def _transpose_body(tt_hbm, rm_hbm, stage_v, pad_v):
    w = lax.axis_index("s") * NC + lax.axis_index("c")
    lane = lax.iota(jnp.int32, 16)
    scat = [lane * ROWP + j * 16 * ROWP for j in range(8)]

    def blk_body(i, c):
        blk = w + i * NW

        @pl.when(blk < NBLK)
        def _():
            pltpu.sync_copy(tt_hbm.at[:, pl.ds(blk * 128, 128)], stage_v)

            def tr_body(tr, cc):
                for j in range(8):
                    vals = stage_v[tr, pl.ds(j * 16, 16)]
                    plsc.store_scatter(pad_v, [scat[j] + tr], vals)
                return cc

            lax.fori_loop(0, TRIL, tr_body, 0)
            pltpu.sync_copy(pad_v, rm_hbm.at[pl.ds(blk * 128 * ROWP,
                                                   128 * ROWP)])

        return c

    lax.fori_loop(0, ABLK_PER_W, blk_body, 0)


def _expand_body(rm_hbm, idx_hbm, out_hbm, idx_v, rows_v, out_v, sem_g, sem_o):
    w = lax.axis_index("s") * NC + lax.axis_index("c")
    c0 = w * CHUNK_PER_W
    lane = lax.iota(jnp.int32, 16)

    pltpu.sync_copy(idx_hbm.at[pl.ds(c0, CHUNK_PER_W), :], idx_v)

    zero16 = jnp.zeros((16,), jnp.float32)

    def zero_body(i, c):
        out_v[pl.ds(i * 16, 16)] = zero16
        return c

    lax.fori_loop(0, CHUNK * OUT_ROW // 16, zero_body, 0, unroll=4)

    def chunk_body(g, c):
        cg = c0 + g
        pltpu.async_copy(rm_hbm.at[idx_v.at[g]], rows_v, sem_g).wait()

        def grp_body(gi, cc):
            r0 = gi * 16
            rvec = lane + r0
            for t in range(TRIL):
                tvec = jnp.full((16,), t, jnp.int32)
                vals = plsc.load_gather(rows_v, [rvec, tvec])
                if _is_diag[t]:
                    vals = jnp.maximum(vals, 1e-4)
                out_v[pl.ds(int(_pos[t]) * 128 + r0, 16)] = vals
            return cc

        lax.fori_loop(0, CHUNK // 16, grp_body, 0)

        s_bb = (cg // 8) * 256 + (cg % 8)
        waits = []
        for t8 in range(32):
            waits.append(pltpu.async_copy(
                out_v.at[pl.ds(t8 * 1024, 1024)],
                out_hbm.at[pl.ds((s_bb + t8 * 8) * 1024, 1024)], sem_o))
        for h in waits:
            h.wait()
        return c

    lax.fori_loop(0, CHUNK_PER_W, chunk_body, 0)


@jax.jit
def kernel(indices, table):
    mesh = plsc.VectorSubcoreMesh(core_axis_name="c", subcore_axis_name="s")

    table_t = table.T
    transpose_call = pl.kernel(
        _transpose_body,
        mesh=mesh,
        compiler_params=pltpu.CompilerParams(
            needs_layout_passes=False, use_tc_tiling_on_sc=True,
            disable_bounds_checks=True),
        out_type=jax.ShapeDtypeStruct((VPAD * ROWP,), jnp.float32),
        scratch_types=[
            pltpu.VMEM((TRIL, 128), jnp.float32),
            pltpu.VMEM((128 * ROWP,), jnp.float32),
        ],
    )
    rm = transpose_call(table_t).reshape(VPAD, ROWP)

    idx2d = indices.T.reshape(NCHUNK_TOTAL, CHUNK)
    expand_call = pl.kernel(
        _expand_body,
        mesh=mesh,
        compiler_params=pltpu.CompilerParams(
            needs_layout_passes=False, use_tc_tiling_on_sc=False),
        out_type=jax.ShapeDtypeStruct((N * OUT_ROW,), jnp.float32),
        scratch_types=[
            pltpu.VMEM((CHUNK_PER_W, CHUNK), jnp.int32),
            pltpu.VMEM((CHUNK, ROWP), jnp.float32),
            pltpu.VMEM((CHUNK * OUT_ROW,), jnp.float32),
            pltpu.SemaphoreType.DMA,
            pltpu.SemaphoreType.DMA,
        ],
    )
    out_flat = expand_call(rm, idx2d)

    t6 = out_flat.reshape(200, 16, 2, 8, 8, 128)
    return t6.transpose(3, 5, 0, 1, 2, 4).reshape(B, S, DIM, DIM)


# speedup vs baseline: 1.5805x; 1.5805x over previous
import functools

import jax
import jax.numpy as jnp
import numpy as np
from jax import lax
from jax.experimental import pallas as pl
from jax.experimental.pallas import tpu as pltpu
from jax.experimental.pallas import tpu_sc as plsc

DIM = 16
TRIL = DIM * (DIM + 1) // 2
ROWP = TRIL + 1
OUT_ROW = DIM * DIM
B, S = 1024, 200
N = B * S

NC, NS = 2, 16
NW = NC * NS

VOCAB = 1000000
VPAD = 1000064
NBLK = VPAD // 128
ABLK_PER_W = -(-NBLK // NW)

CHUNK = 128
NCHUNK_TOTAL = N // CHUNK
CHUNK_PER_W = NCHUNK_TOTAL // NW

_ti, _tj = np.tril_indices(DIM)
_pos = (DIM * _ti + _tj).astype(np.int32)
_is_diag = (_ti == _tj)


_OFFS = tuple(range(0, 113, 16)) + (120,)


def _transpose_body(tt_hbm, rm_hbm, stage_v, pad_v, rows2_v):
    w = lax.axis_index("s") * NC + lax.axis_index("c")
    lane = lax.iota(jnp.int32, 16)
    scat = [lane * ROWP + j * 16 * ROWP for j in range(8)]

    def blk_body(i, c):
        blk = w + i * NW

        @pl.when(blk < NBLK)
        def _():
            pltpu.sync_copy(tt_hbm.at[:, pl.ds(blk * 128, 128)], stage_v)

            def tr_body(tr, cc):
                for j in range(8):
                    vals = stage_v[tr, pl.ds(j * 16, 16)]
                    plsc.store_scatter(pad_v, [scat[j] + tr], vals)
                return cc

            lax.fori_loop(0, TRIL, tr_body, 0)

            def rpk_body(r, cc):
                for off in _OFFS:
                    rows2_v[pl.ds(r * TRIL + off, 16)] = (
                        pad_v[pl.ds(r * ROWP + off, 16)])
                return cc

            lax.fori_loop(0, 128, rpk_body, 0)
            pltpu.sync_copy(rows2_v, rm_hbm.at[pl.ds(blk * 128 * TRIL,
                                                     128 * TRIL)])

        return c

    lax.fori_loop(0, ABLK_PER_W, blk_body, 0)


def _expand_body(rm_hbm, idx_hbm, out_hbm,
                 idx_v, rows_v, pad_v, out_v, sem_g, sem_o):
    w = lax.axis_index("s") * NC + lax.axis_index("c")
    c0 = w * CHUNK_PER_W
    lane = lax.iota(jnp.int32, 16)
    lane137 = lane * ROWP

    pltpu.sync_copy(idx_hbm.at[pl.ds(c0, CHUNK_PER_W), :], idx_v)

    zero16 = jnp.zeros((16,), jnp.float32)

    def zero_body(i, c):
        out_v[pl.ds(i * 16, 16)] = zero16
        return c

    lax.fori_loop(0, CHUNK * OUT_ROW // 16, zero_body, 0, unroll=4)

    def chunk_body(g, c):
        cg = c0 + g
        pltpu.async_copy(rm_hbm.at[idx_v.at[g]], rows_v, sem_g).wait()

        def rpk_body(r, cc):
            for off in _OFFS:
                pad_v[pl.ds(r * ROWP + off, 16)] = rows_v[r, pl.ds(off, 16)]
            return cc

        lax.fori_loop(0, CHUNK, rpk_body, 0)

        def grp_body(gi, cc):
            r0 = gi * 16
            rvec = lane137 + r0 * ROWP
            for t in range(TRIL):
                vals = plsc.load_gather(pad_v, [rvec + t])
                if _is_diag[t]:
                    vals = jnp.maximum(vals, 1e-4)
                out_v[pl.ds(int(_pos[t]) * 128 + r0, 16)] = vals
            return cc

        lax.fori_loop(0, CHUNK // 16, grp_body, 0)

        s_bb = (cg // 8) * 256 + (cg % 8)
        waits = []
        for t8 in range(32):
            waits.append(pltpu.async_copy(
                out_v.at[pl.ds(t8 * 1024, 1024)],
                out_hbm.at[pl.ds((s_bb + t8 * 8) * 1024, 1024)], sem_o))
        for h in waits:
            h.wait()
        return c

    lax.fori_loop(0, CHUNK_PER_W, chunk_body, 0)


@jax.jit
def kernel(indices, table):
    mesh = plsc.VectorSubcoreMesh(core_axis_name="c", subcore_axis_name="s")

    table_t = table.T
    transpose_call = pl.kernel(
        _transpose_body,
        mesh=mesh,
        compiler_params=pltpu.CompilerParams(
            needs_layout_passes=False, use_tc_tiling_on_sc=True,
            disable_bounds_checks=True),
        out_type=jax.ShapeDtypeStruct((VPAD * TRIL,), jnp.float32),
        scratch_types=[
            pltpu.VMEM((TRIL, 128), jnp.float32),
            pltpu.VMEM((128 * ROWP,), jnp.float32),
            pltpu.VMEM((128 * TRIL,), jnp.float32),
        ],
    )
    rm = transpose_call(table_t).reshape(VPAD, TRIL)

    idx2d = indices.T.reshape(NCHUNK_TOTAL, CHUNK)
    expand_call = pl.kernel(
        _expand_body,
        mesh=mesh,
        compiler_params=pltpu.CompilerParams(
            needs_layout_passes=False, use_tc_tiling_on_sc=False),
        out_type=jax.ShapeDtypeStruct((N * OUT_ROW,), jnp.float32),
        scratch_types=[
            pltpu.VMEM((CHUNK_PER_W, CHUNK), jnp.int32),
            pltpu.VMEM((CHUNK, TRIL), jnp.float32),
            pltpu.VMEM((CHUNK * ROWP,), jnp.float32),
            pltpu.VMEM((CHUNK * OUT_ROW,), jnp.float32),
            pltpu.SemaphoreType.DMA,
            pltpu.SemaphoreType.DMA,
        ],
    )
    out_flat = expand_call(rm, idx2d)

    t6 = out_flat.reshape(200, 16, 2, 8, 8, 128)
    return t6.transpose(3, 5, 0, 1, 2, 4).reshape(B, S, DIM, DIM)


# speedup vs baseline: 2.0315x; 1.2854x over previous
import functools

import jax
import jax.numpy as jnp
import numpy as np
from jax import lax
from jax.experimental import pallas as pl
from jax.experimental.pallas import tpu as pltpu
from jax.experimental.pallas import tpu_sc as plsc

DIM = 16
TRIL = DIM * (DIM + 1) // 2
ROWP = TRIL + 1
OUT_ROW = DIM * DIM
B, S = 1024, 200
N = B * S

NC, NS = 2, 16
NW = NC * NS

VOCAB = 1000000
VPAD = 1000064
NBLK = VPAD // 128
ABLK_PER_W = -(-NBLK // NW)

CHUNK = 128
NCHUNK_TOTAL = N // CHUNK
CHUNK_PER_W = NCHUNK_TOTAL // NW

_ti, _tj = np.tril_indices(DIM)
_pos = (DIM * _ti + _tj).astype(np.int32)
_is_diag = (_ti == _tj)


_OFFS = tuple(range(0, 113, 16)) + (120,)


def _transpose_body(tt_hbm, rm_hbm, stage0_v, stage1_v, pad_v,
                    rows20_v, rows21_v, sin0, sin1, sout0, sout1):
    w = lax.axis_index("s") * NC + lax.axis_index("c")
    lane = lax.iota(jnp.int32, 16)
    scat = [lane * ROWP + j * 16 * ROWP for j in range(8)]
    stage = (stage0_v, stage1_v)
    rows2 = (rows20_v, rows21_v)
    sin = (sin0, sin1)
    sout = (sout0, sout1)

    def in_copy(t, b):
        blk = w + t * NW
        return pltpu.make_async_copy(
            tt_hbm.at[:, pl.ds(blk * 128, 128)], stage[b], sin[b])

    def out_copy(t, b):
        blk = w + t * NW
        return pltpu.make_async_copy(
            rows2[b], rm_hbm.at[pl.ds(blk * 128 * TRIL, 128 * TRIL)],
            sout[b])

    def valid(t):
        return (w + t * NW) < NBLK

    for b in range(2):
        in_copy(b, b).start()

    def pair_body(i2, c):
        for b in range(2):
            t = i2 * 2 + b

            @pl.when(valid(t))
            def _():
                in_copy(t, b).wait()

                def tr_body(tr, cc):
                    for j in range(8):
                        vals = stage[b][tr, pl.ds(j * 16, 16)]
                        plsc.store_scatter(pad_v, [scat[j] + tr], vals)
                    return cc

                lax.fori_loop(0, TRIL, tr_body, 0, unroll=2)

            @pl.when(valid(t + 2))
            def _():
                in_copy(t + 2, b).start()

            @pl.when((t >= 2) & valid(t - 2))
            def _():
                out_copy(t - 2, b).wait()

            @pl.when(valid(t))
            def _():
                def rpk_body(r, cc):
                    for off in _OFFS:
                        rows2[b][pl.ds(r * TRIL + off, 16)] = (
                            pad_v[pl.ds(r * ROWP + off, 16)])
                    return cc

                lax.fori_loop(0, 128, rpk_body, 0, unroll=2)
                out_copy(t, b).start()

        return c

    lax.fori_loop(0, (ABLK_PER_W + 1) // 2, pair_body, 0)

    for b in range(2):
        t_last = (ABLK_PER_W + 1) // 2 * 2 - 2 + b

        @pl.when(valid(t_last))
        def _():
            out_copy(t_last, b).wait()


def _expand_body(rm_hbm, idx_hbm, out_hbm, idx_v, rows0_v, rows1_v, pad_v,
                 out0_v, out1_v, sg0, sg1, so0, so1):
    w = lax.axis_index("s") * NC + lax.axis_index("c")
    c0 = w * CHUNK_PER_W
    lane = lax.iota(jnp.int32, 16)
    lane137 = lane * ROWP
    rows = (rows0_v, rows1_v)
    out = (out0_v, out1_v)
    sg = (sg0, sg1)
    so = (so0, so1)

    pltpu.sync_copy(idx_hbm.at[pl.ds(c0, CHUNK_PER_W), :], idx_v)

    def gather(g, b):
        return pltpu.make_async_copy(rm_hbm.at[idx_v.at[g]], rows[b], sg[b])

    def out_copies(g, b):
        cg = c0 + g
        s_bb = (cg // 8) * 256 + (cg % 8)
        return [pltpu.make_async_copy(
            out[b].at[pl.ds(t8 * 1024, 1024)],
            out_hbm.at[pl.ds((s_bb + t8 * 8) * 1024, 1024)], so[b])
            for t8 in range(32)]

    zero16 = jnp.zeros((16,), jnp.float32)

    def zero_body(i, c):
        out0_v[pl.ds(i * 16, 16)] = zero16
        out1_v[pl.ds(i * 16, 16)] = zero16
        return c

    for b in range(2):
        gather(b, b).start()
    lax.fori_loop(0, CHUNK * OUT_ROW // 16, zero_body, 0, unroll=4)

    def pair_body(i2, c):
        for b in range(2):
            g = i2 * 2 + b
            gather(g, b).wait()

            def rpk_body(r, cc):
                for off in _OFFS:
                    pad_v[pl.ds(r * ROWP + off, 16)] = (
                        rows[b][r, pl.ds(off, 16)])
                return cc

            lax.fori_loop(0, CHUNK, rpk_body, 0, unroll=2)

            @pl.when(g + 2 < CHUNK_PER_W)
            def _():
                gather(g + 2, b).start()

            @pl.when(g >= 2)
            def _():
                for h in out_copies(g - 2, b):
                    h.wait()

            def grp_body(gi, cc):
                r0 = gi * 16
                rvec = lane137 + r0 * ROWP
                for t in range(TRIL):
                    vals = plsc.load_gather(pad_v, [rvec + t])
                    if _is_diag[t]:
                        vals = jnp.maximum(vals, 1e-4)
                    out[b][pl.ds(int(_pos[t]) * 128 + r0, 16)] = vals
                return cc

            lax.fori_loop(0, CHUNK // 16, grp_body, 0)
            for h in out_copies(g, b):
                h.start()

        return c

    lax.fori_loop(0, CHUNK_PER_W // 2, pair_body, 0)

    for b in range(2):
        for h in out_copies(CHUNK_PER_W - 2 + b, b):
            h.wait()


@jax.jit
def kernel(indices, table):
    mesh = plsc.VectorSubcoreMesh(core_axis_name="c", subcore_axis_name="s")

    table_t = table.T
    transpose_call = pl.kernel(
        _transpose_body,
        mesh=mesh,
        compiler_params=pltpu.CompilerParams(
            needs_layout_passes=False, use_tc_tiling_on_sc=True,
            disable_bounds_checks=True),
        out_type=jax.ShapeDtypeStruct((VPAD * TRIL,), jnp.float32),
        scratch_types=[
            pltpu.VMEM((TRIL, 128), jnp.float32),
            pltpu.VMEM((TRIL, 128), jnp.float32),
            pltpu.VMEM((128 * ROWP,), jnp.float32),
            pltpu.VMEM((128 * TRIL,), jnp.float32),
            pltpu.VMEM((128 * TRIL,), jnp.float32),
            pltpu.SemaphoreType.DMA,
            pltpu.SemaphoreType.DMA,
            pltpu.SemaphoreType.DMA,
            pltpu.SemaphoreType.DMA,
        ],
    )
    rm = transpose_call(table_t).reshape(VPAD, TRIL)

    idx2d = indices.T.reshape(NCHUNK_TOTAL, CHUNK)
    expand_call = pl.kernel(
        _expand_body,
        mesh=mesh,
        compiler_params=pltpu.CompilerParams(
            needs_layout_passes=False, use_tc_tiling_on_sc=False),
        out_type=jax.ShapeDtypeStruct((N * OUT_ROW,), jnp.float32),
        scratch_types=[
            pltpu.VMEM((CHUNK_PER_W, CHUNK), jnp.int32),
            pltpu.VMEM((CHUNK, TRIL), jnp.float32),
            pltpu.VMEM((CHUNK, TRIL), jnp.float32),
            pltpu.VMEM((CHUNK * ROWP,), jnp.float32),
            pltpu.VMEM((CHUNK * OUT_ROW,), jnp.float32),
            pltpu.VMEM((CHUNK * OUT_ROW,), jnp.float32),
            pltpu.SemaphoreType.DMA,
            pltpu.SemaphoreType.DMA,
            pltpu.SemaphoreType.DMA,
            pltpu.SemaphoreType.DMA,
        ],
    )
    out_flat = expand_call(rm, idx2d)

    t6 = out_flat.reshape(200, 16, 2, 8, 8, 128)
    return t6.transpose(3, 5, 0, 1, 2, 4).reshape(B, S, DIM, DIM)


# speedup vs baseline: 2.0429x; 1.0056x over previous
import functools

import jax
import jax.numpy as jnp
import numpy as np
from jax import lax
from jax.experimental import pallas as pl
from jax.experimental.pallas import tpu as pltpu
from jax.experimental.pallas import tpu_sc as plsc

DIM = 16
TRIL = DIM * (DIM + 1) // 2
ROWP = TRIL + 1
OUT_ROW = DIM * DIM
B, S = 1024, 200
N = B * S

NC, NS = 2, 16
NW = NC * NS

VOCAB = 1000000
VPAD = 1000064
NBLK = VPAD // 128
ABLK_PER_W = -(-NBLK // NW)

CHUNK = 128
NCHUNK_TOTAL = N // CHUNK
CHUNK_PER_W = NCHUNK_TOTAL // NW

_ti, _tj = np.tril_indices(DIM)
_pos = (DIM * _ti + _tj).astype(np.int32)
_is_diag = (_ti == _tj)


_OFFS = tuple(range(0, 113, 16)) + (120,)


def _transpose_body(tt_hbm, rm_hbm, stage0_v, stage1_v, pad_v,
                    rows20_v, rows21_v, sin0, sin1, sout0, sout1):
    w = lax.axis_index("s") * NC + lax.axis_index("c")
    lane = lax.iota(jnp.int32, 16)
    scat = [lane * ROWP + j * 16 * ROWP for j in range(8)]
    stage = (stage0_v, stage1_v)
    rows2 = (rows20_v, rows21_v)
    sin = (sin0, sin1)
    sout = (sout0, sout1)

    def in_copy(t, b):
        blk = w + t * NW
        return pltpu.make_async_copy(
            tt_hbm.at[:, pl.ds(blk * 128, 128)], stage[b], sin[b])

    def out_copy(t, b):
        blk = w + t * NW
        return pltpu.make_async_copy(
            rows2[b], rm_hbm.at[pl.ds(blk * 128 * TRIL, 128 * TRIL)],
            sout[b])

    def valid(t):
        return (w + t * NW) < NBLK

    for b in range(2):
        in_copy(b, b).start()

    def pair_body(i2, c):
        for b in range(2):
            t = i2 * 2 + b

            @pl.when(valid(t))
            def _():
                in_copy(t, b).wait()

                def tr_body(tr, cc):
                    for j in range(8):
                        vals = stage[b][tr, pl.ds(j * 16, 16)]
                        plsc.store_scatter(pad_v, [scat[j] + tr], vals)
                    return cc

                lax.fori_loop(0, TRIL, tr_body, 0, unroll=4)

            @pl.when(valid(t + 2))
            def _():
                in_copy(t + 2, b).start()

            @pl.when((t >= 2) & valid(t - 2))
            def _():
                out_copy(t - 2, b).wait()

            @pl.when(valid(t))
            def _():
                def rpk_body(r, cc):
                    for off in _OFFS:
                        rows2[b][pl.ds(r * TRIL + off, 16)] = (
                            pad_v[pl.ds(r * ROWP + off, 16)])
                    return cc

                lax.fori_loop(0, 128, rpk_body, 0, unroll=4)
                out_copy(t, b).start()

        return c

    lax.fori_loop(0, (ABLK_PER_W + 1) // 2, pair_body, 0)

    for b in range(2):
        t_last = (ABLK_PER_W + 1) // 2 * 2 - 2 + b

        @pl.when(valid(t_last))
        def _():
            out_copy(t_last, b).wait()


def _expand_body(rm_hbm, idx_hbm, out_hbm, idx_v, rows0_v, rows1_v, pad_v,
                 out0_v, out1_v, sg0, sg1, so0, so1):
    w = lax.axis_index("s") * NC + lax.axis_index("c")
    c0 = w * CHUNK_PER_W
    lane = lax.iota(jnp.int32, 16)
    lane137 = lane * ROWP
    rows = (rows0_v, rows1_v)
    out = (out0_v, out1_v)
    sg = (sg0, sg1)
    so = (so0, so1)

    pltpu.sync_copy(idx_hbm.at[pl.ds(c0, CHUNK_PER_W), :], idx_v)

    def gather(g, b):
        return pltpu.make_async_copy(rm_hbm.at[idx_v.at[g]], rows[b], sg[b])

    def out_copies(g, b):
        cg = c0 + g
        s_bb = (cg // 8) * 256 + (cg % 8)
        return [pltpu.make_async_copy(
            out[b].at[pl.ds(t8 * 1024, 1024)],
            out_hbm.at[pl.ds((s_bb + t8 * 8) * 1024, 1024)], so[b])
            for t8 in range(32)]

    zero16 = jnp.zeros((16,), jnp.float32)

    def zero_body(i, c):
        out0_v[pl.ds(i * 16, 16)] = zero16
        out1_v[pl.ds(i * 16, 16)] = zero16
        return c

    for b in range(2):
        gather(b, b).start()
    lax.fori_loop(0, CHUNK * OUT_ROW // 16, zero_body, 0, unroll=4)

    def pair_body(i2, c):
        for b in range(2):
            g = i2 * 2 + b
            gather(g, b).wait()

            def rpk_body(r, cc):
                for off in _OFFS:
                    pad_v[pl.ds(r * ROWP + off, 16)] = (
                        rows[b][r, pl.ds(off, 16)])
                return cc

            lax.fori_loop(0, CHUNK, rpk_body, 0, unroll=4)

            @pl.when(g + 2 < CHUNK_PER_W)
            def _():
                gather(g + 2, b).start()

            @pl.when(g >= 2)
            def _():
                for h in out_copies(g - 2, b):
                    h.wait()

            def grp_body(gi, cc):
                r0 = gi * 16
                rvec = lane137 + r0 * ROWP
                for t in range(TRIL):
                    vals = plsc.load_gather(pad_v, [rvec + t])
                    if _is_diag[t]:
                        vals = jnp.maximum(vals, 1e-4)
                    out[b][pl.ds(int(_pos[t]) * 128 + r0, 16)] = vals
                return cc

            lax.fori_loop(0, CHUNK // 16, grp_body, 0)
            for h in out_copies(g, b):
                h.start()

        return c

    lax.fori_loop(0, CHUNK_PER_W // 2, pair_body, 0)

    for b in range(2):
        for h in out_copies(CHUNK_PER_W - 2 + b, b):
            h.wait()


@jax.jit
def kernel(indices, table):
    mesh = plsc.VectorSubcoreMesh(core_axis_name="c", subcore_axis_name="s")

    table_t = table.T
    transpose_call = pl.kernel(
        _transpose_body,
        mesh=mesh,
        compiler_params=pltpu.CompilerParams(
            needs_layout_passes=False, use_tc_tiling_on_sc=True,
            disable_bounds_checks=True),
        out_type=jax.ShapeDtypeStruct((VPAD * TRIL,), jnp.float32),
        scratch_types=[
            pltpu.VMEM((TRIL, 128), jnp.float32),
            pltpu.VMEM((TRIL, 128), jnp.float32),
            pltpu.VMEM((128 * ROWP,), jnp.float32),
            pltpu.VMEM((128 * TRIL,), jnp.float32),
            pltpu.VMEM((128 * TRIL,), jnp.float32),
            pltpu.SemaphoreType.DMA,
            pltpu.SemaphoreType.DMA,
            pltpu.SemaphoreType.DMA,
            pltpu.SemaphoreType.DMA,
        ],
    )
    rm = transpose_call(table_t).reshape(VPAD, TRIL)

    idx2d = indices.T.reshape(NCHUNK_TOTAL, CHUNK)
    expand_call = pl.kernel(
        _expand_body,
        mesh=mesh,
        compiler_params=pltpu.CompilerParams(
            needs_layout_passes=False, use_tc_tiling_on_sc=False),
        out_type=jax.ShapeDtypeStruct((N * OUT_ROW,), jnp.float32),
        scratch_types=[
            pltpu.VMEM((CHUNK_PER_W, CHUNK), jnp.int32),
            pltpu.VMEM((CHUNK, TRIL), jnp.float32),
            pltpu.VMEM((CHUNK, TRIL), jnp.float32),
            pltpu.VMEM((CHUNK * ROWP,), jnp.float32),
            pltpu.VMEM((CHUNK * OUT_ROW,), jnp.float32),
            pltpu.VMEM((CHUNK * OUT_ROW,), jnp.float32),
            pltpu.SemaphoreType.DMA,
            pltpu.SemaphoreType.DMA,
            pltpu.SemaphoreType.DMA,
            pltpu.SemaphoreType.DMA,
        ],
    )
    out_flat = expand_call(rm, idx2d)

    t6 = out_flat.reshape(200, 16, 2, 8, 8, 128)
    return t6.transpose(3, 5, 0, 1, 2, 4).reshape(B, S, DIM, DIM)
